# Initial kernel scaffold; baseline (speedup 1.0000x reference)
import functools

import jax
import jax.numpy as jnp
from jax import lax
from jax.experimental import pallas as pl
from jax.experimental.pallas import tpu as pltpu
from jax.experimental.pallas import tpu_sc as plsc

VOCAB = 1_000_000
DIM = 64
BATCH = 16384
NEG = 5
LANES = 16
DCHUNKS = DIM // LANES

NUM_CORES = 2
NUM_SUBCORES = 16
NW = NUM_CORES * NUM_SUBCORES
B_PER_W = BATCH // NW
CHUNK = 128
NCHUNKS = B_PER_W // CHUNK


def _sc_scores_kernel(emb_hbm, ctx_hbm, tidx_hbm, cidx_hbm, nidx_hbm,
                      pos_hbm, neg_hbm,
                      tidx_v, cidx_v, nidx_v, trows, crows, nrows,
                      pbuf, nbuf, sem):
    wid = lax.axis_index("s") * NUM_CORES + lax.axis_index("c")

    def do_chunk(c, carry):
        base = wid * B_PER_W + c * CHUNK
        pltpu.sync_copy(tidx_hbm.at[pl.ds(base, CHUNK)], tidx_v)
        pltpu.sync_copy(cidx_hbm.at[pl.ds(base, CHUNK)], cidx_v)
        pltpu.sync_copy(nidx_hbm.at[pl.ds(base * NEG, CHUNK * NEG)], nidx_v)
        cp_t = pltpu.async_copy(emb_hbm.at[tidx_v], trows, sem)
        cp_c = pltpu.async_copy(ctx_hbm.at[cidx_v], crows, sem)
        cp_n = pltpu.async_copy(ctx_hbm.at[nidx_v], nrows, sem)
        cp_t.wait()
        cp_c.wait()
        cp_n.wait()

        def body(b, carry2):
            t = [trows[b, pl.ds(LANES * j, LANES)] for j in range(DCHUNKS)]
            acc = t[0] * crows[b, pl.ds(0, LANES)]
            for j in range(1, DCHUNKS):
                acc = acc + t[j] * crows[b, pl.ds(LANES * j, LANES)]
            pbuf[b] = jnp.sum(acc)
            for k in range(NEG):
                p = b * NEG + k
                acc = t[0] * nrows[p, pl.ds(0, LANES)]
                for j in range(1, DCHUNKS):
                    acc = acc + t[j] * nrows[p, pl.ds(LANES * j, LANES)]
                nbuf[p] = jnp.sum(acc)
            return carry2

        lax.fori_loop(0, CHUNK, body, 0)
        pltpu.sync_copy(pbuf, pos_hbm.at[pl.ds(base, CHUNK)])
        pltpu.sync_copy(nbuf, neg_hbm.at[pl.ds(base * NEG, CHUNK * NEG)])
        return carry

    lax.fori_loop(0, NCHUNKS, do_chunk, 0)


_sc_scores = functools.partial(
    pl.kernel,
    mesh=plsc.VectorSubcoreMesh(core_axis_name="c", subcore_axis_name="s"),
    out_type=[
        jax.ShapeDtypeStruct((BATCH,), jnp.float32),
        jax.ShapeDtypeStruct((BATCH * NEG,), jnp.float32),
    ],
    scratch_types=[
        pltpu.VMEM((CHUNK,), jnp.int32),
        pltpu.VMEM((CHUNK,), jnp.int32),
        pltpu.VMEM((CHUNK * NEG,), jnp.int32),
        pltpu.VMEM((CHUNK, DIM), jnp.float32),
        pltpu.VMEM((CHUNK, DIM), jnp.float32),
        pltpu.VMEM((CHUNK * NEG, DIM), jnp.float32),
        pltpu.VMEM((CHUNK,), jnp.float32),
        pltpu.VMEM((CHUNK * NEG,), jnp.float32),
        pltpu.SemaphoreType.DMA,
    ],
)(_sc_scores_kernel)


def _tc_loss_kernel(pos_ref, neg_ref, out_ref):
    pos = pos_ref[...]
    neg = -neg_ref[...]
    pos_ls = jnp.minimum(pos, 0.0) - jnp.log1p(jnp.exp(-jnp.abs(pos)))
    neg_ls = jnp.minimum(neg, 0.0) - jnp.log1p(jnp.exp(-jnp.abs(neg)))
    out_ref[0, 0] = -(jnp.sum(pos_ls) + jnp.sum(neg_ls)) / BATCH


def _tc_loss(pos_s, neg_s):
    return pl.pallas_call(
        _tc_loss_kernel,
        out_shape=jax.ShapeDtypeStruct((1, 1), jnp.float32),
        out_specs=pl.BlockSpec(memory_space=pltpu.SMEM),
    )(pos_s, neg_s)


def kernel(emb_table, ctx_table, target_words, context_words, negative_samples):
    tidx = target_words.astype(jnp.int32)
    cidx = context_words.astype(jnp.int32)
    nidx = negative_samples.astype(jnp.int32).reshape(-1)
    pos_s, neg_s = _sc_scores(emb_table, ctx_table, tidx, cidx, nidx)
    loss = _tc_loss(pos_s.reshape(128, 128), neg_s.reshape(640, 128))
    return loss[0, 0]


# speedup vs baseline: 2.6273x; 2.6273x over previous
import functools

import jax
import jax.numpy as jnp
from jax import lax
from jax.experimental import pallas as pl
from jax.experimental.pallas import tpu as pltpu
from jax.experimental.pallas import tpu_sc as plsc

VOCAB = 1_000_000
DIM = 64
BATCH = 16384
NEG = 5
LANES = 16

ROWS2 = VOCAB // 2
WIDE = 2 * DIM

NUM_CORES = 2
NUM_SUBCORES = 16
NW = NUM_CORES * NUM_SUBCORES
B_PER_W = BATCH // NW
CHUNK = 128
NCHUNKS = B_PER_W // CHUNK


def _sc_scores_kernel(emb_hbm, ctx_hbm, tidx_hbm, cidx_hbm, nidx_hbm,
                      pos_hbm, neg_hbm,
                      tidx_v, cidx_v, nidx_v, th_v, ch_v, nh_v,
                      trows, crows, nrows, pbuf, nbuf, sem):
    wid = lax.axis_index("s") * NUM_CORES + lax.axis_index("c")

    def do_chunk(c, carry):
        base = wid * B_PER_W + c * CHUNK
        pltpu.sync_copy(tidx_hbm.at[pl.ds(base, CHUNK)], tidx_v)
        pltpu.sync_copy(cidx_hbm.at[pl.ds(base, CHUNK)], cidx_v)
        pltpu.sync_copy(nidx_hbm.at[pl.ds(base * NEG, CHUNK * NEG)], nidx_v)

        def halve(i, carry2):
            s = pl.ds(i * LANES, LANES)
            th_v[s] = tidx_v[s] // 2
            ch_v[s] = cidx_v[s] // 2
            return carry2

        lax.fori_loop(0, CHUNK // LANES, halve, 0)

        def halve_n(i, carry2):
            s = pl.ds(i * LANES, LANES)
            nh_v[s] = nidx_v[s] // 2
            return carry2

        lax.fori_loop(0, CHUNK * NEG // LANES, halve_n, 0)

        cp_t = pltpu.async_copy(emb_hbm.at[th_v], trows, sem)
        cp_c = pltpu.async_copy(ctx_hbm.at[ch_v], crows, sem)
        cp_n = pltpu.async_copy(ctx_hbm.at[nh_v], nrows, sem)
        cp_t.wait()
        cp_c.wait()
        cp_n.wait()

        def body(g, carry2):
            b0 = g * LANES
            li = lax.iota(jnp.int32, LANES)
            rt = b0 + li
            rn = [rt * NEG + k for k in range(NEG)]
            tb = (tidx_v[pl.ds(b0, LANES)] & 1) * DIM
            cb = (cidx_v[pl.ds(b0, LANES)] & 1) * DIM
            nb = [(plsc.load_gather(nidx_v, [rn[k]]) & 1) * DIM
                  for k in range(NEG)]
            zero = jnp.zeros((LANES,), jnp.float32)
            acc_p = zero
            acc_n = [zero] * NEG
            for d in range(DIM):
                tv = plsc.load_gather(trows, [rt, tb + d])
                cv = plsc.load_gather(crows, [rt, cb + d])
                acc_p = acc_p + tv * cv
                for k in range(NEG):
                    nv = plsc.load_gather(nrows, [rn[k], nb[k] + d])
                    acc_n[k] = acc_n[k] + tv * nv
            pbuf[pl.ds(b0, LANES)] = acc_p
            for k in range(NEG):
                plsc.store_scatter(nbuf, [rn[k]], acc_n[k])
            return carry2

        lax.fori_loop(0, CHUNK // LANES, body, 0)
        pltpu.sync_copy(pbuf, pos_hbm.at[pl.ds(base, CHUNK)])
        pltpu.sync_copy(nbuf, neg_hbm.at[pl.ds(base * NEG, CHUNK * NEG)])
        return carry

    lax.fori_loop(0, NCHUNKS, do_chunk, 0)


_sc_scores = functools.partial(
    pl.kernel,
    mesh=plsc.VectorSubcoreMesh(core_axis_name="c", subcore_axis_name="s"),
    compiler_params=pltpu.CompilerParams(
        needs_layout_passes=False, use_tc_tiling_on_sc=False),
    out_type=[
        jax.ShapeDtypeStruct((BATCH,), jnp.float32),
        jax.ShapeDtypeStruct((BATCH * NEG,), jnp.float32),
    ],
    scratch_types=[
        pltpu.VMEM((CHUNK,), jnp.int32),
        pltpu.VMEM((CHUNK,), jnp.int32),
        pltpu.VMEM((CHUNK * NEG,), jnp.int32),
        pltpu.VMEM((CHUNK,), jnp.int32),
        pltpu.VMEM((CHUNK,), jnp.int32),
        pltpu.VMEM((CHUNK * NEG,), jnp.int32),
        pltpu.VMEM((CHUNK, WIDE), jnp.float32),
        pltpu.VMEM((CHUNK, WIDE), jnp.float32),
        pltpu.VMEM((CHUNK * NEG, WIDE), jnp.float32),
        pltpu.VMEM((CHUNK,), jnp.float32),
        pltpu.VMEM((CHUNK * NEG,), jnp.float32),
        pltpu.SemaphoreType.DMA,
    ],
)(_sc_scores_kernel)


_RB = 8192
_RBLKS = (VOCAB + _RB - 1) // _RB


def _tc_repack_kernel(x_ref, o_ref, z_ref):
    z_ref[...] = jnp.transpose(x_ref[...])
    o_ref[:, 0:DIM] = z_ref[0::2, :]
    o_ref[:, DIM:WIDE] = z_ref[1::2, :]


def _tc_repack(t):
    return pl.pallas_call(
        _tc_repack_kernel,
        grid=(_RBLKS,),
        in_specs=[pl.BlockSpec((DIM, _RB), lambda g: (0, g))],
        out_specs=pl.BlockSpec((_RB // 2, WIDE), lambda g: (g, 0)),
        out_shape=jax.ShapeDtypeStruct((ROWS2, WIDE), jnp.float32),
        scratch_shapes=[pltpu.VMEM((_RB, DIM), jnp.float32)],
    )(t)


def _tc_loss_kernel(pos_ref, neg_ref, out_ref):
    pos = pos_ref[...]
    neg = -neg_ref[...]
    pos_ls = jnp.minimum(pos, 0.0) - jnp.log1p(jnp.exp(-jnp.abs(pos)))
    neg_ls = jnp.minimum(neg, 0.0) - jnp.log1p(jnp.exp(-jnp.abs(neg)))
    out_ref[0, 0] = -(jnp.sum(pos_ls) + jnp.sum(neg_ls)) / BATCH


def _tc_loss(pos_s, neg_s):
    return pl.pallas_call(
        _tc_loss_kernel,
        out_shape=jax.ShapeDtypeStruct((1, 1), jnp.float32),
        out_specs=pl.BlockSpec(memory_space=pltpu.SMEM),
    )(pos_s, neg_s)


def kernel(emb_table, ctx_table, target_words, context_words, negative_samples):
    emb_r = _tc_repack(emb_table.T)
    ctx_r = _tc_repack(ctx_table.T)
    tidx = target_words.astype(jnp.int32)
    cidx = context_words.astype(jnp.int32)
    nidx = negative_samples.astype(jnp.int32).reshape(-1)
    pos_s, neg_s = _sc_scores(emb_r, ctx_r, tidx, cidx, nidx)
    loss = _tc_loss(pos_s.reshape(128, 128), neg_s.reshape(640, 128))
    return loss[0, 0]


# speedup vs baseline: 2.8913x; 1.1005x over previous
import functools

import jax
import jax.numpy as jnp
from jax import lax
from jax.experimental import pallas as pl
from jax.experimental.pallas import tpu as pltpu
from jax.experimental.pallas import tpu_sc as plsc

VOCAB = 1_000_000
DIM = 64
BATCH = 16384
NEG = 5
LANES = 16

ROWS2 = VOCAB // 2
WIDE = 2 * DIM

NUM_CORES = 2
NUM_SUBCORES = 16
NW = NUM_CORES * NUM_SUBCORES
B_PER_W = BATCH // NW
CHUNK = 128
NCHUNKS = B_PER_W // CHUNK


def _sc_scores_kernel(emb_hbm, ctx_hbm, tidx_hbm, cidx_hbm, nidx_hbm,
                      pos_hbm, neg_hbm,
                      tidx_v, cidx_v, nidx_v, th_v, ch_v, nh_v,
                      trows, crows, nrows, pbuf, nbuf, sem):
    wid = lax.axis_index("s") * NUM_CORES + lax.axis_index("c")

    def do_chunk(c, carry):
        base = wid * B_PER_W + c * CHUNK
        pltpu.sync_copy(tidx_hbm.at[pl.ds(base, CHUNK)], tidx_v)
        pltpu.sync_copy(cidx_hbm.at[pl.ds(base, CHUNK)], cidx_v)
        pltpu.sync_copy(nidx_hbm.at[pl.ds(base * NEG, CHUNK * NEG)], nidx_v)

        def halve(i, carry2):
            s = pl.ds(i * LANES, LANES)
            th_v[s] = _packed_row(tidx_v[s])
            ch_v[s] = _packed_row(cidx_v[s])
            return carry2

        lax.fori_loop(0, CHUNK // LANES, halve, 0)

        def halve_n(i, carry2):
            s = pl.ds(i * LANES, LANES)
            nh_v[s] = _packed_row(nidx_v[s])
            return carry2

        lax.fori_loop(0, CHUNK * NEG // LANES, halve_n, 0)

        cp_t = pltpu.async_copy(emb_hbm.at[th_v], trows, sem)
        cp_c = pltpu.async_copy(ctx_hbm.at[ch_v], crows, sem)
        cp_n = pltpu.async_copy(ctx_hbm.at[nh_v], nrows, sem)
        cp_t.wait()
        cp_c.wait()
        cp_n.wait()

        def body(g, carry2):
            b0 = g * LANES
            li = lax.iota(jnp.int32, LANES)
            rt = b0 + li
            rn = [rt * NEG + k for k in range(NEG)]
            tb = _packed_base(tidx_v[pl.ds(b0, LANES)])
            cb = _packed_base(cidx_v[pl.ds(b0, LANES)])
            nb = [_packed_base(plsc.load_gather(nidx_v, [rn[k]]))
                  for k in range(NEG)]
            zero = jnp.zeros((LANES,), jnp.float32)
            acc_p = zero
            acc_n = [zero] * NEG
            for d in range(DIM):
                tv = plsc.load_gather(trows, [rt, tb + d])
                cv = plsc.load_gather(crows, [rt, cb + d])
                acc_p = acc_p + tv * cv
                for k in range(NEG):
                    nv = plsc.load_gather(nrows, [rn[k], nb[k] + d])
                    acc_n[k] = acc_n[k] + tv * nv
            pbuf[pl.ds(b0, LANES)] = acc_p
            for k in range(NEG):
                plsc.store_scatter(nbuf, [rn[k]], acc_n[k])
            return carry2

        lax.fori_loop(0, CHUNK // LANES, body, 0)
        pltpu.sync_copy(pbuf, pos_hbm.at[pl.ds(base, CHUNK)])
        pltpu.sync_copy(nbuf, neg_hbm.at[pl.ds(base * NEG, CHUNK * NEG)])
        return carry

    lax.fori_loop(0, NCHUNKS, do_chunk, 0)


_sc_scores = functools.partial(
    pl.kernel,
    mesh=plsc.VectorSubcoreMesh(core_axis_name="c", subcore_axis_name="s"),
    compiler_params=pltpu.CompilerParams(
        needs_layout_passes=False, use_tc_tiling_on_sc=False),
    out_type=[
        jax.ShapeDtypeStruct((BATCH,), jnp.float32),
        jax.ShapeDtypeStruct((BATCH * NEG,), jnp.float32),
    ],
    scratch_types=[
        pltpu.VMEM((CHUNK,), jnp.int32),
        pltpu.VMEM((CHUNK,), jnp.int32),
        pltpu.VMEM((CHUNK * NEG,), jnp.int32),
        pltpu.VMEM((CHUNK,), jnp.int32),
        pltpu.VMEM((CHUNK,), jnp.int32),
        pltpu.VMEM((CHUNK * NEG,), jnp.int32),
        pltpu.VMEM((CHUNK, WIDE), jnp.float32),
        pltpu.VMEM((CHUNK, WIDE), jnp.float32),
        pltpu.VMEM((CHUNK * NEG, WIDE), jnp.float32),
        pltpu.VMEM((CHUNK,), jnp.float32),
        pltpu.VMEM((CHUNK * NEG,), jnp.float32),
        pltpu.SemaphoreType.DMA,
    ],
)(_sc_scores_kernel)


_RB = 8192
_HALF = _RB // 2
_RBLKS = (VOCAB + _RB - 1) // _RB
PROWS = _RBLKS * _HALF


def _tc_repack_kernel(x_ref, o_ref):
    zl = jnp.transpose(x_ref[:, 0:_HALF])
    zr = jnp.transpose(x_ref[:, _HALF:_RB])
    o_ref[...] = jnp.concatenate([zl, zr], axis=1)


def _tc_repack(t):
    return pl.pallas_call(
        _tc_repack_kernel,
        grid=(_RBLKS,),
        in_specs=[pl.BlockSpec((DIM, _RB), lambda g: (0, g))],
        out_specs=pl.BlockSpec((_HALF, WIDE), lambda g: (g, 0)),
        out_shape=jax.ShapeDtypeStruct((PROWS, WIDE), jnp.float32),
        compiler_params=pltpu.CompilerParams(
            fuse_transposed_lhs_in_matmul=True),
    )(t)


def _packed_row(v):
    return ((v >> 13) << 12) | (v & 4095)


def _packed_base(v):
    return ((v >> 12) & 1) * DIM


def _tc_loss_kernel(pos_ref, neg_ref, out_ref):
    pos = pos_ref[...]
    neg = -neg_ref[...]
    pos_ls = jnp.minimum(pos, 0.0) - jnp.log1p(jnp.exp(-jnp.abs(pos)))
    neg_ls = jnp.minimum(neg, 0.0) - jnp.log1p(jnp.exp(-jnp.abs(neg)))
    out_ref[0, 0] = -(jnp.sum(pos_ls) + jnp.sum(neg_ls)) / BATCH


def _tc_loss(pos_s, neg_s):
    return pl.pallas_call(
        _tc_loss_kernel,
        out_shape=jax.ShapeDtypeStruct((1, 1), jnp.float32),
        out_specs=pl.BlockSpec(memory_space=pltpu.SMEM),
    )(pos_s, neg_s)


def kernel(emb_table, ctx_table, target_words, context_words, negative_samples):
    emb_r = _tc_repack(emb_table.T)
    ctx_r = _tc_repack(ctx_table.T)
    tidx = target_words.astype(jnp.int32)
    cidx = context_words.astype(jnp.int32)
    nidx = negative_samples.astype(jnp.int32).reshape(-1)
    pos_s, neg_s = _sc_scores(emb_r, ctx_r, tidx, cidx, nidx)
    loss = _tc_loss(pos_s.reshape(128, 128), neg_s.reshape(640, 128))
    return loss[0, 0]


# speedup vs baseline: 3.5414x; 1.2248x over previous
import functools

import jax
import jax.numpy as jnp
from jax import lax
from jax.experimental import pallas as pl
from jax.experimental.pallas import tpu as pltpu
from jax.experimental.pallas import tpu_sc as plsc

VOCAB = 1_000_000
DIM = 64
BATCH = 16384
NEG = 5
LANES = 16

ROWS2 = VOCAB // 2
WIDE = 2 * DIM

NUM_CORES = 2
NUM_SUBCORES = 16
NW = NUM_CORES * NUM_SUBCORES
B_PER_W = BATCH // NW
CHUNK = 128
NCHUNKS = B_PER_W // CHUNK


def _sc_scores_kernel(emb_hbm, ctx_hbm, tidx_hbm, cidx_hbm, nidx_hbm,
                      pos_hbm, neg_hbm,
                      tidx_v, cidx_v, nidx_v, th_v, ch_v, nh_v,
                      trows, crows, nrows, pbuf, nbuf, sem):
    wid = lax.axis_index("s") * NUM_CORES + lax.axis_index("c")

    def do_chunk(c, carry):
        base = wid * B_PER_W + c * CHUNK
        pltpu.sync_copy(tidx_hbm.at[pl.ds(base, CHUNK)], tidx_v)
        pltpu.sync_copy(cidx_hbm.at[pl.ds(base, CHUNK)], cidx_v)
        pltpu.sync_copy(nidx_hbm.at[pl.ds(base * NEG, CHUNK * NEG)], nidx_v)

        def halve(i, carry2):
            s = pl.ds(i * LANES, LANES)
            th_v[s] = _packed_row(tidx_v[s])
            ch_v[s] = _packed_row(cidx_v[s])
            return carry2

        lax.fori_loop(0, CHUNK // LANES, halve, 0)

        def halve_n(i, carry2):
            s = pl.ds(i * LANES, LANES)
            nh_v[s] = _packed_row(nidx_v[s])
            return carry2

        lax.fori_loop(0, CHUNK * NEG // LANES, halve_n, 0)

        cp_t = pltpu.async_copy(emb_hbm.at[th_v], trows, sem)
        cp_c = pltpu.async_copy(ctx_hbm.at[ch_v], crows, sem)
        cp_n = pltpu.async_copy(ctx_hbm.at[nh_v], nrows, sem)
        cp_t.wait()
        cp_c.wait()
        cp_n.wait()

        def body(g, carry2):
            b0 = g * LANES
            li = lax.iota(jnp.int32, LANES)
            rt = b0 + li
            rn = [rt * NEG + k for k in range(NEG)]
            tb = _packed_base(tidx_v[pl.ds(b0, LANES)])
            cb = _packed_base(cidx_v[pl.ds(b0, LANES)])
            nb = [_packed_base(plsc.load_gather(nidx_v, [rn[k]]))
                  for k in range(NEG)]
            zero = jnp.zeros((LANES,), jnp.float32)
            acc_p = zero
            acc_n = [zero] * NEG
            for d in range(DIM):
                tv = plsc.load_gather(trows, [rt, tb + d])
                cv = plsc.load_gather(crows, [rt, cb + d])
                acc_p = acc_p + tv * cv
                for k in range(NEG):
                    nv = plsc.load_gather(nrows, [rn[k], nb[k] + d])
                    acc_n[k] = acc_n[k] + tv * nv
            pbuf[pl.ds(b0, LANES)] = acc_p
            for k in range(NEG):
                plsc.store_scatter(nbuf, [rn[k]], acc_n[k])
            return carry2

        lax.fori_loop(0, CHUNK // LANES, body, 0)
        pltpu.sync_copy(pbuf, pos_hbm.at[pl.ds(base, CHUNK)])
        pltpu.sync_copy(nbuf, neg_hbm.at[pl.ds(base * NEG, CHUNK * NEG)])
        return carry

    lax.fori_loop(0, NCHUNKS, do_chunk, 0)


_sc_scores = functools.partial(
    pl.kernel,
    mesh=plsc.VectorSubcoreMesh(core_axis_name="c", subcore_axis_name="s"),
    compiler_params=pltpu.CompilerParams(
        needs_layout_passes=False, use_tc_tiling_on_sc=False),
    out_type=[
        jax.ShapeDtypeStruct((BATCH,), jnp.float32),
        jax.ShapeDtypeStruct((BATCH * NEG,), jnp.float32),
    ],
    scratch_types=[
        pltpu.VMEM((CHUNK,), jnp.int32),
        pltpu.VMEM((CHUNK,), jnp.int32),
        pltpu.VMEM((CHUNK * NEG,), jnp.int32),
        pltpu.VMEM((CHUNK,), jnp.int32),
        pltpu.VMEM((CHUNK,), jnp.int32),
        pltpu.VMEM((CHUNK * NEG,), jnp.int32),
        pltpu.VMEM((CHUNK, WIDE), jnp.float32),
        pltpu.VMEM((CHUNK, WIDE), jnp.float32),
        pltpu.VMEM((CHUNK * NEG, WIDE), jnp.float32),
        pltpu.VMEM((CHUNK,), jnp.float32),
        pltpu.VMEM((CHUNK * NEG,), jnp.float32),
        pltpu.SemaphoreType.DMA,
    ],
)(_sc_scores_kernel)


_RB = 8192
_HALF = _RB // 2
_RBLKS = (VOCAB + _RB - 1) // _RB
PROWS = _RBLKS * _HALF


def _tc_repack_kernel(x_ref, o_ref):
    x2 = jnp.concatenate([x_ref[:, 0:_HALF], x_ref[:, _HALF:_RB]], axis=0)
    o_ref[...] = jnp.transpose(x2)


def _tc_repack(t):
    return pl.pallas_call(
        _tc_repack_kernel,
        grid=(_RBLKS,),
        in_specs=[pl.BlockSpec((DIM, _RB), lambda g: (0, g))],
        out_specs=pl.BlockSpec((_HALF, WIDE), lambda g: (g, 0)),
        out_shape=jax.ShapeDtypeStruct((PROWS, WIDE), jnp.float32),
        compiler_params=pltpu.CompilerParams(
            fuse_transposed_lhs_in_matmul=True),
    )(t)


def _packed_row(v):
    return ((v >> 13) << 12) | (v & 4095)


def _packed_base(v):
    return ((v >> 12) & 1) * DIM


def _tc_loss_kernel(pos_ref, neg_ref, out_ref):
    pos = pos_ref[...]
    neg = -neg_ref[...]
    pos_ls = jnp.minimum(pos, 0.0) - jnp.log1p(jnp.exp(-jnp.abs(pos)))
    neg_ls = jnp.minimum(neg, 0.0) - jnp.log1p(jnp.exp(-jnp.abs(neg)))
    out_ref[0, 0] = -(jnp.sum(pos_ls) + jnp.sum(neg_ls)) / BATCH


def _tc_loss(pos_s, neg_s):
    return pl.pallas_call(
        _tc_loss_kernel,
        out_shape=jax.ShapeDtypeStruct((1, 1), jnp.float32),
        out_specs=pl.BlockSpec(memory_space=pltpu.SMEM),
    )(pos_s, neg_s)


def kernel(emb_table, ctx_table, target_words, context_words, negative_samples):
    emb_r = _tc_repack(emb_table.T)
    ctx_r = _tc_repack(ctx_table.T)
    tidx = target_words.astype(jnp.int32)
    cidx = context_words.astype(jnp.int32)
    nidx = negative_samples.astype(jnp.int32).reshape(-1)
    pos_s, neg_s = _sc_scores(emb_r, ctx_r, tidx, cidx, nidx)
    loss = _tc_loss(pos_s.reshape(128, 128), neg_s.reshape(640, 128))
    return loss[0, 0]


# speedup vs baseline: 3.9329x; 1.1105x over previous
import functools

import jax
import jax.numpy as jnp
from jax import lax
from jax.experimental import pallas as pl
from jax.experimental.pallas import tpu as pltpu
from jax.experimental.pallas import tpu_sc as plsc

VOCAB = 1_000_000
DIM = 64
BATCH = 16384
NEG = 5
LANES = 16

ROWS2 = VOCAB // 2
WIDE = 2 * DIM

NUM_CORES = 2
NUM_SUBCORES = 16
NW = NUM_CORES * NUM_SUBCORES
B_PER_W = BATCH // NW
CHUNK = 128
NCHUNKS = B_PER_W // CHUNK


def _sc_scores_kernel(emb_hbm, ctx_hbm, tidx_hbm, cidx_hbm, nidx_hbm,
                      pos_hbm, neg_hbm,
                      tidx_v, cidx_v, nidx_v, th_v, ch_v, nh_v,
                      trows, crows, nrows, pbuf, nbuf, sem):
    wid = lax.axis_index("s") * NUM_CORES + lax.axis_index("c")

    def do_chunk(c, carry):
        base = wid * B_PER_W + c * CHUNK
        pltpu.sync_copy(tidx_hbm.at[pl.ds(base, CHUNK)], tidx_v)
        pltpu.sync_copy(cidx_hbm.at[pl.ds(base, CHUNK)], cidx_v)
        pltpu.sync_copy(nidx_hbm.at[pl.ds(base * NEG, CHUNK * NEG)], nidx_v)

        def halve(i, carry2):
            s = pl.ds(i * LANES, LANES)
            th_v[s] = _packed_row(tidx_v[s])
            ch_v[s] = _packed_row(cidx_v[s])
            return carry2

        lax.fori_loop(0, CHUNK // LANES, halve, 0)

        def halve_n(i, carry2):
            s = pl.ds(i * LANES, LANES)
            nh_v[s] = _packed_row(nidx_v[s])
            return carry2

        lax.fori_loop(0, CHUNK * NEG // LANES, halve_n, 0)

        cp_t = pltpu.async_copy(emb_hbm.at[th_v], trows, sem)
        cp_c = pltpu.async_copy(ctx_hbm.at[ch_v], crows, sem)
        cp_n = pltpu.async_copy(ctx_hbm.at[nh_v], nrows, sem)
        cp_t.wait()
        cp_c.wait()
        cp_n.wait()

        def body(g, carry2):
            b0 = g * LANES
            li = lax.iota(jnp.int32, LANES)
            rt = b0 + li
            rn = [rt * NEG + k for k in range(NEG)]
            tb = _packed_base(tidx_v[pl.ds(b0, LANES)])
            cb = _packed_base(cidx_v[pl.ds(b0, LANES)])
            nb = [_packed_base(plsc.load_gather(nidx_v, [rn[k]]))
                  for k in range(NEG)]
            zero = jnp.zeros((LANES,), jnp.float32)
            acc_p = zero
            acc_n = [zero] * NEG
            for d in range(DIM):
                tv = plsc.load_gather(trows, [rt, tb + d])
                cv = plsc.load_gather(crows, [rt, cb + d])
                acc_p = acc_p + tv * cv
                for k in range(NEG):
                    nv = plsc.load_gather(nrows, [rn[k], nb[k] + d])
                    acc_n[k] = acc_n[k] + tv * nv
            pbuf[pl.ds(b0, LANES)] = acc_p
            for k in range(NEG):
                plsc.store_scatter(nbuf, [rn[k]], acc_n[k])
            return carry2

        lax.fori_loop(0, CHUNK // LANES, body, 0)
        pltpu.sync_copy(pbuf, pos_hbm.at[pl.ds(base, CHUNK)])
        pltpu.sync_copy(nbuf, neg_hbm.at[pl.ds(base * NEG, CHUNK * NEG)])
        return carry

    lax.fori_loop(0, NCHUNKS, do_chunk, 0)


_sc_scores = functools.partial(
    pl.kernel,
    mesh=plsc.VectorSubcoreMesh(core_axis_name="c", subcore_axis_name="s"),
    compiler_params=pltpu.CompilerParams(
        needs_layout_passes=False, use_tc_tiling_on_sc=False),
    out_type=[
        jax.ShapeDtypeStruct((BATCH,), jnp.float32),
        jax.ShapeDtypeStruct((BATCH * NEG,), jnp.float32),
    ],
    scratch_types=[
        pltpu.VMEM((CHUNK,), jnp.int32),
        pltpu.VMEM((CHUNK,), jnp.int32),
        pltpu.VMEM((CHUNK * NEG,), jnp.int32),
        pltpu.VMEM((CHUNK,), jnp.int32),
        pltpu.VMEM((CHUNK,), jnp.int32),
        pltpu.VMEM((CHUNK * NEG,), jnp.int32),
        pltpu.VMEM((CHUNK, WIDE), jnp.float32),
        pltpu.VMEM((CHUNK, WIDE), jnp.float32),
        pltpu.VMEM((CHUNK * NEG, WIDE), jnp.float32),
        pltpu.VMEM((CHUNK,), jnp.float32),
        pltpu.VMEM((CHUNK * NEG,), jnp.float32),
        pltpu.SemaphoreType.DMA,
    ],
)(_sc_scores_kernel)


_RB = 16384
_HALF = _RB // 2
_SH = _RB.bit_length() - 1
_RBLKS = (VOCAB + _RB - 1) // _RB
PROWS = _RBLKS * _HALF


def _tc_repack_kernel(x_ref, o_ref):
    x2 = jnp.concatenate([x_ref[:, 0:_HALF], x_ref[:, _HALF:_RB]], axis=0)
    o_ref[...] = jnp.transpose(x2)


def _tc_repack(t):
    return pl.pallas_call(
        _tc_repack_kernel,
        grid=(_RBLKS,),
        in_specs=[pl.BlockSpec((DIM, _RB), lambda g: (0, g))],
        out_specs=pl.BlockSpec((_HALF, WIDE), lambda g: (g, 0)),
        out_shape=jax.ShapeDtypeStruct((PROWS, WIDE), jnp.float32),
        compiler_params=pltpu.CompilerParams(
            fuse_transposed_lhs_in_matmul=True),
    )(t)


def _packed_row(v):
    return ((v >> _SH) << (_SH - 1)) | (v & (_HALF - 1))


def _packed_base(v):
    return ((v >> (_SH - 1)) & 1) * DIM


def _tc_loss_kernel(pos_ref, neg_ref, out_ref):
    pos = pos_ref[...]
    neg = -neg_ref[...]
    pos_ls = jnp.minimum(pos, 0.0) - jnp.log1p(jnp.exp(-jnp.abs(pos)))
    neg_ls = jnp.minimum(neg, 0.0) - jnp.log1p(jnp.exp(-jnp.abs(neg)))
    out_ref[0, 0] = -(jnp.sum(pos_ls) + jnp.sum(neg_ls)) / BATCH


def _tc_loss(pos_s, neg_s):
    return pl.pallas_call(
        _tc_loss_kernel,
        out_shape=jax.ShapeDtypeStruct((1, 1), jnp.float32),
        out_specs=pl.BlockSpec(memory_space=pltpu.SMEM),
    )(pos_s, neg_s)


def kernel(emb_table, ctx_table, target_words, context_words, negative_samples):
    emb_r = _tc_repack(emb_table.T)
    ctx_r = _tc_repack(ctx_table.T)
    tidx = target_words.astype(jnp.int32)
    cidx = context_words.astype(jnp.int32)
    nidx = negative_samples.astype(jnp.int32).reshape(-1)
    pos_s, neg_s = _sc_scores(emb_r, ctx_r, tidx, cidx, nidx)
    loss = _tc_loss(pos_s.reshape(128, 128), neg_s.reshape(640, 128))
    return loss[0, 0]


# speedup vs baseline: 4.0142x; 1.0207x over previous
import functools

import jax
import jax.numpy as jnp
from jax import lax
from jax.experimental import pallas as pl
from jax.experimental.pallas import tpu as pltpu
from jax.experimental.pallas import tpu_sc as plsc

VOCAB = 1_000_000
DIM = 64
BATCH = 16384
NEG = 5
LANES = 16

ROWS2 = VOCAB // 2
WIDE = 2 * DIM

NUM_CORES = 2
NUM_SUBCORES = 16
NW = NUM_CORES * NUM_SUBCORES
B_PER_W = BATCH // NW
CHUNK = 128
NCHUNKS = B_PER_W // CHUNK


def _sc_scores_kernel(emb_hbm, ctx_hbm, tidx_hbm, cidx_hbm, nidx_hbm,
                      pos_hbm, neg_hbm,
                      tidx_v, cidx_v, nidx_v, th_v, ch_v, nh_v,
                      trows, crows, nrows, pbuf, nbuf, sem):
    wid = lax.axis_index("s") * NUM_CORES + lax.axis_index("c")

    def do_chunk(c, carry):
        base = wid * B_PER_W + c * CHUNK
        pltpu.sync_copy(tidx_hbm.at[pl.ds(base, CHUNK)], tidx_v)
        pltpu.sync_copy(cidx_hbm.at[pl.ds(base, CHUNK)], cidx_v)
        pltpu.sync_copy(nidx_hbm.at[pl.ds(base * NEG, CHUNK * NEG)], nidx_v)

        def halve(i, carry2):
            s = pl.ds(i * LANES, LANES)
            th_v[s] = _packed_row(tidx_v[s])
            ch_v[s] = _packed_row(cidx_v[s])
            return carry2

        lax.fori_loop(0, CHUNK // LANES, halve, 0)

        def halve_n(i, carry2):
            s = pl.ds(i * LANES, LANES)
            nh_v[s] = _packed_row(nidx_v[s])
            return carry2

        lax.fori_loop(0, CHUNK * NEG // LANES, halve_n, 0)

        cp_t = pltpu.async_copy(emb_hbm.at[th_v], trows, sem)
        cp_c = pltpu.async_copy(ctx_hbm.at[ch_v], crows, sem)
        cp_n = pltpu.async_copy(ctx_hbm.at[nh_v], nrows, sem)
        cp_t.wait()
        cp_c.wait()
        cp_n.wait()

        def body(g, carry2):
            b0 = g * LANES
            li = lax.iota(jnp.int32, LANES)
            rt = b0 + li
            rn = [rt * NEG + k for k in range(NEG)]
            tb = _packed_base(tidx_v[pl.ds(b0, LANES)])
            cb = _packed_base(cidx_v[pl.ds(b0, LANES)])
            nb = [_packed_base(plsc.load_gather(nidx_v, [rn[k]]))
                  for k in range(NEG)]
            zero = jnp.zeros((LANES,), jnp.float32)
            acc_p = zero
            acc_n = [zero] * NEG
            for d in range(DIM):
                tv = plsc.load_gather(trows, [rt, tb + d])
                cv = plsc.load_gather(crows, [rt, cb + d])
                acc_p = acc_p + tv * cv
                for k in range(NEG):
                    nv = plsc.load_gather(nrows, [rn[k], nb[k] + d])
                    acc_n[k] = acc_n[k] + tv * nv
            pbuf[pl.ds(b0, LANES)] = acc_p
            for k in range(NEG):
                plsc.store_scatter(nbuf, [rn[k]], acc_n[k])
            return carry2

        lax.fori_loop(0, CHUNK // LANES, body, 0)
        pltpu.sync_copy(pbuf, pos_hbm.at[pl.ds(base, CHUNK)])
        pltpu.sync_copy(nbuf, neg_hbm.at[pl.ds(base * NEG, CHUNK * NEG)])
        return carry

    lax.fori_loop(0, NCHUNKS, do_chunk, 0)


_sc_scores = functools.partial(
    pl.kernel,
    mesh=plsc.VectorSubcoreMesh(core_axis_name="c", subcore_axis_name="s"),
    compiler_params=pltpu.CompilerParams(
        needs_layout_passes=False, use_tc_tiling_on_sc=False),
    out_type=[
        jax.ShapeDtypeStruct((BATCH,), jnp.float32),
        jax.ShapeDtypeStruct((BATCH * NEG,), jnp.float32),
    ],
    scratch_types=[
        pltpu.VMEM((CHUNK,), jnp.int32),
        pltpu.VMEM((CHUNK,), jnp.int32),
        pltpu.VMEM((CHUNK * NEG,), jnp.int32),
        pltpu.VMEM((CHUNK,), jnp.int32),
        pltpu.VMEM((CHUNK,), jnp.int32),
        pltpu.VMEM((CHUNK * NEG,), jnp.int32),
        pltpu.VMEM((CHUNK, WIDE), jnp.float32),
        pltpu.VMEM((CHUNK, WIDE), jnp.float32),
        pltpu.VMEM((CHUNK * NEG, WIDE), jnp.float32),
        pltpu.VMEM((CHUNK,), jnp.float32),
        pltpu.VMEM((CHUNK * NEG,), jnp.float32),
        pltpu.SemaphoreType.DMA,
    ],
)(_sc_scores_kernel)


_RB = 32768
_HALF = _RB // 2
_SH = _RB.bit_length() - 1
_RBLKS = (VOCAB + _RB - 1) // _RB
PROWS = _RBLKS * _HALF


def _tc_repack_kernel(x_ref, o_ref):
    x2 = jnp.concatenate([x_ref[:, 0:_HALF], x_ref[:, _HALF:_RB]], axis=0)
    o_ref[...] = jnp.transpose(x2)


def _tc_repack(t):
    return pl.pallas_call(
        _tc_repack_kernel,
        grid=(_RBLKS,),
        in_specs=[pl.BlockSpec((DIM, _RB), lambda g: (0, g))],
        out_specs=pl.BlockSpec((_HALF, WIDE), lambda g: (g, 0)),
        out_shape=jax.ShapeDtypeStruct((PROWS, WIDE), jnp.float32),
        compiler_params=pltpu.CompilerParams(
            fuse_transposed_lhs_in_matmul=True),
    )(t)


def _packed_row(v):
    return ((v >> _SH) << (_SH - 1)) | (v & (_HALF - 1))


def _packed_base(v):
    return ((v >> (_SH - 1)) & 1) * DIM


def _tc_loss_kernel(pos_ref, neg_ref, out_ref):
    pos = pos_ref[...]
    neg = -neg_ref[...]
    pos_ls = jnp.minimum(pos, 0.0) - jnp.log1p(jnp.exp(-jnp.abs(pos)))
    neg_ls = jnp.minimum(neg, 0.0) - jnp.log1p(jnp.exp(-jnp.abs(neg)))
    out_ref[0, 0] = -(jnp.sum(pos_ls) + jnp.sum(neg_ls)) / BATCH


def _tc_loss(pos_s, neg_s):
    return pl.pallas_call(
        _tc_loss_kernel,
        out_shape=jax.ShapeDtypeStruct((1, 1), jnp.float32),
        out_specs=pl.BlockSpec(memory_space=pltpu.SMEM),
    )(pos_s, neg_s)


def kernel(emb_table, ctx_table, target_words, context_words, negative_samples):
    emb_r = _tc_repack(emb_table.T)
    ctx_r = _tc_repack(ctx_table.T)
    tidx = target_words.astype(jnp.int32)
    cidx = context_words.astype(jnp.int32)
    nidx = negative_samples.astype(jnp.int32).reshape(-1)
    pos_s, neg_s = _sc_scores(emb_r, ctx_r, tidx, cidx, nidx)
    loss = _tc_loss(pos_s.reshape(128, 128), neg_s.reshape(640, 128))
    return loss[0, 0]


# speedup vs baseline: 4.0988x; 1.0211x over previous
import functools

import jax
import jax.numpy as jnp
from jax import lax
from jax.experimental import pallas as pl
from jax.experimental.pallas import tpu as pltpu
from jax.experimental.pallas import tpu_sc as plsc

VOCAB = 1_000_000
DIM = 64
BATCH = 16384
NEG = 5
LANES = 16

ROWS2 = VOCAB // 2
WIDE = 2 * DIM

NUM_CORES = 2
NUM_SUBCORES = 16
NW = NUM_CORES * NUM_SUBCORES
B_PER_W = BATCH // NW
CHUNK = 64
NCHUNKS = B_PER_W // CHUNK


def _sc_scores_kernel(emb_hbm, ctx_hbm, tidx_hbm, cidx_hbm, nidx_hbm,
                      pos_hbm, neg_hbm,
                      tidx0, cidx0, nidx0, th0, ch0, nh0, tr0, cr0, nr0,
                      tidx1, cidx1, nidx1, th1, ch1, nh1, tr1, cr1, nr1,
                      pbuf, nbuf, sem0, sem1):
    wid = lax.axis_index("s") * NUM_CORES + lax.axis_index("c")
    slots = [
        (tidx0, cidx0, nidx0, th0, ch0, nh0, tr0, cr0, nr0, sem0),
        (tidx1, cidx1, nidx1, th1, ch1, nh1, tr1, cr1, nr1, sem1),
    ]

    def stage(c, slot):
        tidx_v, cidx_v, nidx_v, th_v, ch_v, nh_v, trows, crows, nrows, sem = slot

        @pl.when(c < NCHUNKS)
        def _():
            base = wid * B_PER_W + c * CHUNK
            pltpu.sync_copy(tidx_hbm.at[pl.ds(base, CHUNK)], tidx_v)
            pltpu.sync_copy(cidx_hbm.at[pl.ds(base, CHUNK)], cidx_v)
            pltpu.sync_copy(nidx_hbm.at[pl.ds(base * NEG, CHUNK * NEG)],
                            nidx_v)

            def halve(i, carry2):
                s = pl.ds(i * LANES, LANES)
                th_v[s] = _packed_row(tidx_v[s])
                ch_v[s] = _packed_row(cidx_v[s])
                return carry2

            lax.fori_loop(0, CHUNK // LANES, halve, 0)

            def halve_n(i, carry2):
                s = pl.ds(i * LANES, LANES)
                nh_v[s] = _packed_row(nidx_v[s])
                return carry2

            lax.fori_loop(0, CHUNK * NEG // LANES, halve_n, 0)

            pltpu.async_copy(emb_hbm.at[th_v], trows, sem)
            pltpu.async_copy(ctx_hbm.at[ch_v], crows, sem)
            pltpu.async_copy(ctx_hbm.at[nh_v], nrows, sem)

    def wait_and_compute(c, slot):
        tidx_v, cidx_v, nidx_v, th_v, ch_v, nh_v, trows, crows, nrows, sem = slot
        pltpu.make_async_copy(emb_hbm.at[th_v], trows, sem).wait()
        pltpu.make_async_copy(ctx_hbm.at[ch_v], crows, sem).wait()
        pltpu.make_async_copy(ctx_hbm.at[nh_v], nrows, sem).wait()

        def body(g, carry2):
            b0 = g * LANES
            li = lax.iota(jnp.int32, LANES)
            rt = b0 + li
            rn = [rt * NEG + k for k in range(NEG)]
            tb = _packed_base(tidx_v[pl.ds(b0, LANES)])
            cb = _packed_base(cidx_v[pl.ds(b0, LANES)])
            nb = [_packed_base(plsc.load_gather(nidx_v, [rn[k]]))
                  for k in range(NEG)]
            zero = jnp.zeros((LANES,), jnp.float32)
            acc_p = zero
            acc_n = [zero] * NEG
            for d in range(DIM):
                tv = plsc.load_gather(trows, [rt, tb + d])
                cv = plsc.load_gather(crows, [rt, cb + d])
                acc_p = acc_p + tv * cv
                for k in range(NEG):
                    nv = plsc.load_gather(nrows, [rn[k], nb[k] + d])
                    acc_n[k] = acc_n[k] + tv * nv
            pbuf[pl.ds(b0, LANES)] = acc_p
            for k in range(NEG):
                plsc.store_scatter(nbuf, [rn[k]], acc_n[k])
            return carry2

        lax.fori_loop(0, CHUNK // LANES, body, 0)
        base = wid * B_PER_W + c * CHUNK
        pltpu.sync_copy(pbuf, pos_hbm.at[pl.ds(base, CHUNK)])
        pltpu.sync_copy(nbuf, neg_hbm.at[pl.ds(base * NEG, CHUNK * NEG)])

    stage(0, slots[0])

    def outer(j, carry):
        c0 = 2 * j
        stage(c0 + 1, slots[1])
        wait_and_compute(c0, slots[0])
        stage(c0 + 2, slots[0])
        wait_and_compute(c0 + 1, slots[1])
        return carry

    lax.fori_loop(0, NCHUNKS // 2, outer, 0)


def _slot_scratch():
    return [
        pltpu.VMEM((CHUNK,), jnp.int32),
        pltpu.VMEM((CHUNK,), jnp.int32),
        pltpu.VMEM((CHUNK * NEG,), jnp.int32),
        pltpu.VMEM((CHUNK,), jnp.int32),
        pltpu.VMEM((CHUNK,), jnp.int32),
        pltpu.VMEM((CHUNK * NEG,), jnp.int32),
        pltpu.VMEM((CHUNK, WIDE), jnp.float32),
        pltpu.VMEM((CHUNK, WIDE), jnp.float32),
        pltpu.VMEM((CHUNK * NEG, WIDE), jnp.float32),
    ]


_sc_scores = functools.partial(
    pl.kernel,
    mesh=plsc.VectorSubcoreMesh(core_axis_name="c", subcore_axis_name="s"),
    compiler_params=pltpu.CompilerParams(
        needs_layout_passes=False, use_tc_tiling_on_sc=False),
    out_type=[
        jax.ShapeDtypeStruct((BATCH,), jnp.float32),
        jax.ShapeDtypeStruct((BATCH * NEG,), jnp.float32),
    ],
    scratch_types=(
        _slot_scratch() + _slot_scratch() + [
            pltpu.VMEM((CHUNK,), jnp.float32),
            pltpu.VMEM((CHUNK * NEG,), jnp.float32),
            pltpu.SemaphoreType.DMA,
            pltpu.SemaphoreType.DMA,
        ]
    ),
)(_sc_scores_kernel)


_RB = 32768
_HALF = _RB // 2
_SH = _RB.bit_length() - 1
_RBLKS = (VOCAB + _RB - 1) // _RB
PROWS = _RBLKS * _HALF


def _tc_repack_kernel(x_ref, o_ref):
    x2 = jnp.concatenate([x_ref[:, 0:_HALF], x_ref[:, _HALF:_RB]], axis=0)
    o_ref[...] = jnp.transpose(x2)


def _tc_repack(t):
    return pl.pallas_call(
        _tc_repack_kernel,
        grid=(_RBLKS,),
        in_specs=[pl.BlockSpec((DIM, _RB), lambda g: (0, g))],
        out_specs=pl.BlockSpec((_HALF, WIDE), lambda g: (g, 0)),
        out_shape=jax.ShapeDtypeStruct((PROWS, WIDE), jnp.float32),
        compiler_params=pltpu.CompilerParams(
            fuse_transposed_lhs_in_matmul=True),
    )(t)


def _packed_row(v):
    return ((v >> _SH) << (_SH - 1)) | (v & (_HALF - 1))


def _packed_base(v):
    return ((v >> (_SH - 1)) & 1) * DIM


def _tc_loss_kernel(pos_ref, neg_ref, out_ref):
    pos = pos_ref[...]
    neg = -neg_ref[...]
    pos_ls = jnp.minimum(pos, 0.0) - jnp.log1p(jnp.exp(-jnp.abs(pos)))
    neg_ls = jnp.minimum(neg, 0.0) - jnp.log1p(jnp.exp(-jnp.abs(neg)))
    out_ref[0, 0] = -(jnp.sum(pos_ls) + jnp.sum(neg_ls)) / BATCH


def _tc_loss(pos_s, neg_s):
    return pl.pallas_call(
        _tc_loss_kernel,
        out_shape=jax.ShapeDtypeStruct((1, 1), jnp.float32),
        out_specs=pl.BlockSpec(memory_space=pltpu.SMEM),
    )(pos_s, neg_s)


def kernel(emb_table, ctx_table, target_words, context_words, negative_samples):
    emb_r = _tc_repack(emb_table.T)
    ctx_r = _tc_repack(ctx_table.T)
    tidx = target_words.astype(jnp.int32)
    cidx = context_words.astype(jnp.int32)
    nidx = negative_samples.astype(jnp.int32).reshape(-1)
    pos_s, neg_s = _sc_scores(emb_r, ctx_r, tidx, cidx, nidx)
    loss = _tc_loss(pos_s.reshape(128, 128), neg_s.reshape(640, 128))
    return loss[0, 0]


# speedup vs baseline: 4.9301x; 1.2028x over previous
import functools

import jax
import jax.numpy as jnp
from jax import lax
from jax.experimental import pallas as pl
from jax.experimental.pallas import tpu as pltpu
from jax.experimental.pallas import tpu_sc as plsc

VOCAB = 1_000_000
DIM = 64
BATCH = 16384
NEG = 5
LANES = 16

ROWS2 = VOCAB // 2
WIDE = 2 * DIM

NUM_CORES = 2
NUM_SUBCORES = 16
NW = NUM_CORES * NUM_SUBCORES
B_PER_W = BATCH // NW
CHUNK = 64
NCHUNKS = B_PER_W // CHUNK


def _sc_scores_kernel(emb_hbm, ctx_hbm, tidx_hbm, cidx_hbm, nidx_hbm,
                      pos_hbm, neg_hbm,
                      tidx0, cidx0, nidx0, th0, ch0, nh0, tr0, cr0, nr0,
                      tidx1, cidx1, nidx1, th1, ch1, nh1, tr1, cr1, nr1,
                      pbuf, nbuf, sem0, sem1):
    wid = lax.axis_index("s") * NUM_CORES + lax.axis_index("c")
    slots = [
        (tidx0, cidx0, nidx0, th0, ch0, nh0, tr0, cr0, nr0, sem0),
        (tidx1, cidx1, nidx1, th1, ch1, nh1, tr1, cr1, nr1, sem1),
    ]

    def stage(c, slot):
        tidx_v, cidx_v, nidx_v, th_v, ch_v, nh_v, trows, crows, nrows, sem = slot

        @pl.when(c < NCHUNKS)
        def _():
            base = wid * B_PER_W + c * CHUNK
            pltpu.sync_copy(tidx_hbm.at[pl.ds(base, CHUNK)], tidx_v)
            pltpu.sync_copy(cidx_hbm.at[pl.ds(base, CHUNK)], cidx_v)
            pltpu.sync_copy(nidx_hbm.at[pl.ds(base * NEG, CHUNK * NEG)],
                            nidx_v)

            def halve(i, carry2):
                s = pl.ds(i * LANES, LANES)
                th_v[s] = _packed_row(tidx_v[s])
                ch_v[s] = _packed_row(cidx_v[s])
                return carry2

            lax.fori_loop(0, CHUNK // LANES, halve, 0)

            def halve_n(i, carry2):
                s = pl.ds(i * LANES, LANES)
                nh_v[s] = _packed_row(nidx_v[s])
                return carry2

            lax.fori_loop(0, CHUNK * NEG // LANES, halve_n, 0)

            pltpu.async_copy(emb_hbm.at[th_v], trows, sem)
            pltpu.async_copy(ctx_hbm.at[ch_v], crows, sem)
            pltpu.async_copy(ctx_hbm.at[nh_v], nrows, sem)

    def wait_and_compute(c, slot):
        tidx_v, cidx_v, nidx_v, th_v, ch_v, nh_v, trows, crows, nrows, sem = slot
        pltpu.make_async_copy(emb_hbm.at[th_v], trows, sem).wait()
        pltpu.make_async_copy(ctx_hbm.at[ch_v], crows, sem).wait()
        pltpu.make_async_copy(ctx_hbm.at[nh_v], nrows, sem).wait()

        def body(g, carry2):
            b0 = g * LANES
            li = lax.iota(jnp.int32, LANES)
            rt = b0 + li
            rn = [rt * NEG + k for k in range(NEG)]
            ti = tidx_v[pl.ds(b0, LANES)]
            ci = cidx_v[pl.ds(b0, LANES)]
            ni = [plsc.load_gather(nidx_v, [rn[k]]) for k in range(NEG)]
            tb, tsh = _packed_base(ti), _packed_sh(ti)
            cb, csh = _packed_base(ci), _packed_sh(ci)
            nb = [_packed_base(x) for x in ni]
            nsh = [_packed_sh(x) for x in ni]
            zero = jnp.zeros((LANES,), jnp.float32)
            acc_p = zero
            acc_n = [zero] * NEG
            for d in range(DIM):
                tv = _bf16_hi(plsc.load_gather(trows, [rt, tb + d]), tsh)
                cv = _bf16_hi(plsc.load_gather(crows, [rt, cb + d]), csh)
                acc_p = acc_p + tv * cv
                for k in range(NEG):
                    nv = _bf16_hi(
                        plsc.load_gather(nrows, [rn[k], nb[k] + d]), nsh[k])
                    acc_n[k] = acc_n[k] + tv * nv
            pbuf[pl.ds(b0, LANES)] = acc_p
            for k in range(NEG):
                plsc.store_scatter(nbuf, [rn[k]], acc_n[k])
            return carry2

        lax.fori_loop(0, CHUNK // LANES, body, 0)
        base = wid * B_PER_W + c * CHUNK
        pltpu.sync_copy(pbuf, pos_hbm.at[pl.ds(base, CHUNK)])
        pltpu.sync_copy(nbuf, neg_hbm.at[pl.ds(base * NEG, CHUNK * NEG)])

    stage(0, slots[0])

    def outer(j, carry):
        c0 = 2 * j
        stage(c0 + 1, slots[1])
        wait_and_compute(c0, slots[0])
        stage(c0 + 2, slots[0])
        wait_and_compute(c0 + 1, slots[1])
        return carry

    lax.fori_loop(0, NCHUNKS // 2, outer, 0)


def _slot_scratch():
    return [
        pltpu.VMEM((CHUNK,), jnp.int32),
        pltpu.VMEM((CHUNK,), jnp.int32),
        pltpu.VMEM((CHUNK * NEG,), jnp.int32),
        pltpu.VMEM((CHUNK,), jnp.int32),
        pltpu.VMEM((CHUNK,), jnp.int32),
        pltpu.VMEM((CHUNK * NEG,), jnp.int32),
        pltpu.VMEM((CHUNK, WIDE), jnp.int32),
        pltpu.VMEM((CHUNK, WIDE), jnp.int32),
        pltpu.VMEM((CHUNK * NEG, WIDE), jnp.int32),
    ]


_sc_scores = functools.partial(
    pl.kernel,
    mesh=plsc.VectorSubcoreMesh(core_axis_name="c", subcore_axis_name="s"),
    compiler_params=pltpu.CompilerParams(
        needs_layout_passes=False, use_tc_tiling_on_sc=False),
    out_type=[
        jax.ShapeDtypeStruct((BATCH,), jnp.float32),
        jax.ShapeDtypeStruct((BATCH * NEG,), jnp.float32),
    ],
    scratch_types=(
        _slot_scratch() + _slot_scratch() + [
            pltpu.VMEM((CHUNK,), jnp.float32),
            pltpu.VMEM((CHUNK * NEG,), jnp.float32),
            pltpu.SemaphoreType.DMA,
            pltpu.SemaphoreType.DMA,
        ]
    ),
)(_sc_scores_kernel)


_RB = 32768
_HALF = _RB // 2
_SH = _RB.bit_length() - 1
_RBLKS = (VOCAB + _RB - 1) // _RB
PROWS = _RBLKS * _HALF


def _tc_repack_kernel(x_ref, o_ref):
    x2 = jnp.concatenate([x_ref[:, 0:_HALF], x_ref[:, _HALF:_RB]], axis=0)
    z = jnp.transpose(x2)
    zl = z[0:_HALF // 2]
    zh = z[_HALF // 2:_HALF]
    lo = lax.convert_element_type(
        lax.bitcast_convert_type(zl.astype(jnp.bfloat16), jnp.uint16),
        jnp.uint32)
    hi = lax.convert_element_type(
        lax.bitcast_convert_type(zh.astype(jnp.bfloat16), jnp.uint16),
        jnp.uint32)
    o_ref[...] = lax.bitcast_convert_type((hi << 16) | lo, jnp.int32)


def _tc_repack(t):
    return pl.pallas_call(
        _tc_repack_kernel,
        grid=(_RBLKS,),
        in_specs=[pl.BlockSpec((DIM, _RB), lambda g: (0, g))],
        out_specs=pl.BlockSpec((_HALF // 2, WIDE), lambda g: (g, 0)),
        out_shape=jax.ShapeDtypeStruct((PROWS // 2, WIDE), jnp.int32),
    )(t)


def _packed_row(v):
    return ((v >> _SH) << (_SH - 2)) | (v & (_HALF // 2 - 1))


def _packed_sh(v):
    return ((v >> (_SH - 2)) & 1) * 16


def _packed_base(v):
    return ((v >> (_SH - 1)) & 1) * DIM


def _bf16_hi(w, sh):
    return plsc.bitcast((w >> sh) << 16, jnp.float32)


def _tc_loss_kernel(pos_ref, neg_ref, out_ref):
    pos = pos_ref[...]
    neg = -neg_ref[...]
    pos_ls = jnp.minimum(pos, 0.0) - jnp.log1p(jnp.exp(-jnp.abs(pos)))
    neg_ls = jnp.minimum(neg, 0.0) - jnp.log1p(jnp.exp(-jnp.abs(neg)))
    out_ref[0, 0] = -(jnp.sum(pos_ls) + jnp.sum(neg_ls)) / BATCH


def _tc_loss(pos_s, neg_s):
    return pl.pallas_call(
        _tc_loss_kernel,
        out_shape=jax.ShapeDtypeStruct((1, 1), jnp.float32),
        out_specs=pl.BlockSpec(memory_space=pltpu.SMEM),
    )(pos_s, neg_s)


def kernel(emb_table, ctx_table, target_words, context_words, negative_samples):
    emb_r = _tc_repack(emb_table.T)
    ctx_r = _tc_repack(ctx_table.T)
    tidx = target_words.astype(jnp.int32)
    cidx = context_words.astype(jnp.int32)
    nidx = negative_samples.astype(jnp.int32).reshape(-1)
    pos_s, neg_s = _sc_scores(emb_r, ctx_r, tidx, cidx, nidx)
    loss = _tc_loss(pos_s.reshape(128, 128), neg_s.reshape(640, 128))
    return loss[0, 0]
